# Initial kernel scaffold; baseline (speedup 1.0000x reference)
import jax
import jax.numpy as jnp
from jax.experimental import pallas as pl


def kernel(inputs, Wg, We, be):
    raise NotImplementedError("write your pallas kernel here")


# speedup vs baseline: 2.1022x; 2.1022x over previous
import functools

import jax
import jax.numpy as jnp
from jax.experimental import pallas as pl
from jax.experimental.pallas import tpu as pltpu

E = 8
K = 2
D = 768
T = 8192
BT = 512


def _moe_body(x_ref, wgt_ref, wet_ref, be_ref, out_ref):
    x = x_ref[...]
    logits = jnp.dot(x, wgt_ref[...], preferred_element_type=jnp.float32)
    iota = jax.lax.broadcasted_iota(jnp.int32, (BT, E), 1)
    v1 = jnp.max(logits, axis=1, keepdims=True)
    i1 = jnp.min(jnp.where(logits == v1, iota, E), axis=1, keepdims=True)
    oh1 = iota == i1
    masked = jnp.where(oh1, -jnp.inf, logits)
    v2 = jnp.max(masked, axis=1, keepdims=True)
    i2 = jnp.min(jnp.where(masked == v2, iota, E), axis=1, keepdims=True)
    oh2 = iota == i2
    t = jnp.exp(v2 - v1)
    denom = 1.0 + t
    w = jnp.where(oh1, 1.0 / denom, 0.0) + jnp.where(oh2, t / denom, 0.0)

    xb = x.astype(jnp.bfloat16)
    acc = jnp.zeros((BT, D), dtype=jnp.float32)
    for e in range(E):
        y = jnp.dot(xb, wet_ref[e], preferred_element_type=jnp.float32)
        acc = acc + w[:, e : e + 1] * (y + be_ref[e][None, :])
    out_ref[...] = acc


@jax.jit
def _moe(inputs, wgt, wet, be):
    grid = T // BT
    return pl.pallas_call(
        _moe_body,
        grid=(grid,),
        in_specs=[
            pl.BlockSpec((BT, D), lambda i: (i, 0)),
            pl.BlockSpec((D, E), lambda i: (0, 0)),
            pl.BlockSpec((E, D, D), lambda i: (0, 0, 0)),
            pl.BlockSpec((E, D), lambda i: (0, 0)),
        ],
        out_specs=pl.BlockSpec((BT, D), lambda i: (i, 0)),
        out_shape=jax.ShapeDtypeStruct((T, D), jnp.float32),
    )(inputs, wgt, wet, be)


def kernel(inputs, Wg, We, be):
    wgt = Wg.T
    wet = jnp.swapaxes(We, 1, 2).astype(jnp.bfloat16)
    return _moe(inputs, wgt, wet, be)
